# Initial kernel scaffold; baseline (speedup 1.0000x reference)
import jax
import jax.numpy as jnp
from jax.experimental import pallas as pl


def kernel(x, pe):
    raise NotImplementedError("write your pallas kernel here")


# speedup vs baseline: 1.7267x; 1.7267x over previous
import jax
import jax.numpy as jnp
from jax.experimental import pallas as pl
from jax.experimental.pallas import tpu as pltpu

_S_BLK = 512


def _add_pe_kernel(x_ref, pe_ref, o_ref):
    o_ref[...] = x_ref[...] + pe_ref[...][None, :, :]


def kernel(x, pe):
    batch, seq, d_model = x.shape
    grid = (seq // _S_BLK,)
    return pl.pallas_call(
        _add_pe_kernel,
        grid=grid,
        in_specs=[
            pl.BlockSpec((batch, _S_BLK, d_model), lambda i: (0, i, 0)),
            pl.BlockSpec((_S_BLK, d_model), lambda i: (i, 0)),
        ],
        out_specs=pl.BlockSpec((batch, _S_BLK, d_model), lambda i: (0, i, 0)),
        out_shape=jax.ShapeDtypeStruct((batch, seq, d_model), x.dtype),
        compiler_params=pltpu.CompilerParams(
            dimension_semantics=("arbitrary",),
        ),
    )(x, pe)
